# initial kernel scaffold (unmeasured)
import jax
import jax.numpy as jnp
from jax import lax
from jax.experimental import pallas as pl
from jax.experimental.pallas import tpu as pltpu

N_DEV = 16
N_LAYERS = 3


def kernel(x, Win0, Wout0, Win1, Wout1, Win2, Wout2):
    b, d = x.shape

    def body(
        x_ref,
        win0_ref,
        wout0_ref,
        win1_ref,
        wout1_ref,
        win2_ref,
        wout2_ref,
        out_ref,
        mine_ref,
        comm_ref,
        send_sems,
        recv_sems,
    ):
        my = lax.axis_index("i")
        wins = [win0_ref, win1_ref, win2_ref]
        wouts = [wout0_ref, wout1_ref, wout2_ref]

        acc = x_ref[...]
        for layer in range(N_LAYERS):
            h = jnp.maximum(
                jnp.dot(acc, wins[layer][...], preferred_element_type=jnp.float32),
                0.0,
            )
            partial = jnp.dot(
                h, wouts[layer][...], preferred_element_type=jnp.float32
            )
            mine_ref[...] = partial

            sends = []
            for o in range(1, N_DEV):
                tgt = lax.rem(my + o, N_DEV)
                rdma = pltpu.make_async_remote_copy(
                    src_ref=mine_ref,
                    dst_ref=comm_ref.at[layer, o],
                    send_sem=send_sems.at[layer, o],
                    recv_sem=recv_sems.at[layer, o],
                    device_id=(tgt,),
                    device_id_type=pl.DeviceIdType.MESH,
                )
                rdma.start()
                sends.append(rdma)

            acc = partial
            for o in range(1, N_DEV):
                recv = pltpu.make_async_remote_copy(
                    src_ref=comm_ref.at[layer, o],
                    dst_ref=comm_ref.at[layer, o],
                    send_sem=send_sems.at[layer, o],
                    recv_sem=recv_sems.at[layer, o],
                    device_id=(my,),
                    device_id_type=pl.DeviceIdType.MESH,
                )
                recv.wait_recv()
                acc = acc + comm_ref[layer, o]

            for rdma in sends:
                rdma.wait_send()

        out_ref[...] = acc

    return pl.pallas_call(
        body,
        out_shape=jax.ShapeDtypeStruct((b, d), jnp.float32),
        in_specs=[pl.BlockSpec(memory_space=pltpu.VMEM)] * 7,
        out_specs=pl.BlockSpec(memory_space=pltpu.VMEM),
        scratch_shapes=[
            pltpu.VMEM((b, d), jnp.float32),
            pltpu.VMEM((N_LAYERS, N_DEV, b, d), jnp.float32),
            pltpu.SemaphoreType.DMA((N_LAYERS, N_DEV)),
            pltpu.SemaphoreType.DMA((N_LAYERS, N_DEV)),
        ],
        compiler_params=pltpu.CompilerParams(collective_id=0),
    )(x, Win0, Wout0, Win1, Wout1, Win2, Wout2)


# baseline (device time: 52628 ns/iter reference)
import jax
import jax.numpy as jnp
from jax import lax
from jax.experimental import pallas as pl
from jax.experimental.pallas import tpu as pltpu

N_DEV = 16
N_LAYERS = 3


def kernel(x, Win0, Wout0, Win1, Wout1, Win2, Wout2):
    b, d = x.shape

    def body(
        x_ref,
        win0_ref,
        wout0_ref,
        win1_ref,
        wout1_ref,
        win2_ref,
        wout2_ref,
        out_ref,
        mine_ref,
        comm_ref,
        send_sems,
        recv_sems,
    ):
        my = lax.axis_index("i")
        wins = [win0_ref, win1_ref, win2_ref]
        wouts = [wout0_ref, wout1_ref, wout2_ref]

        acc = x_ref[...]
        for layer in range(N_LAYERS):
            h = jnp.maximum(
                jnp.dot(acc, wins[layer][...], preferred_element_type=jnp.float32),
                0.0,
            )
            partial = jnp.dot(
                h, wouts[layer][...], preferred_element_type=jnp.float32
            )
            mine_ref[...] = partial

            sends = []
            for o in range(1, N_DEV):
                tgt = lax.rem(my + o, N_DEV)
                rdma = pltpu.make_async_remote_copy(
                    src_ref=mine_ref,
                    dst_ref=comm_ref.at[layer, o],
                    send_sem=send_sems.at[layer, o],
                    recv_sem=recv_sems.at[layer, o],
                    device_id=(tgt,),
                    device_id_type=pl.DeviceIdType.MESH,
                )
                rdma.start()
                sends.append(rdma)

            acc = partial
            for o in range(1, N_DEV):
                recv = pltpu.make_async_remote_copy(
                    src_ref=comm_ref.at[layer, o],
                    dst_ref=comm_ref.at[layer, o],
                    send_sem=send_sems.at[layer, o],
                    recv_sem=recv_sems.at[layer, o],
                    device_id=(my,),
                    device_id_type=pl.DeviceIdType.MESH,
                )
                recv.wait_recv()
                acc = acc + comm_ref[layer, o]

            for rdma in sends:
                rdma.wait_send()

        out_ref[...] = acc

    return pl.pallas_call(
        body,
        out_shape=jax.ShapeDtypeStruct((b, d), jnp.float32),
        in_specs=[pl.BlockSpec(memory_space=pltpu.VMEM)] * 7,
        out_specs=pl.BlockSpec(memory_space=pltpu.VMEM),
        scratch_shapes=[
            pltpu.VMEM((b, d), jnp.float32),
            pltpu.VMEM((N_LAYERS, N_DEV, b, d), jnp.float32),
            pltpu.SemaphoreType.DMA((N_LAYERS, N_DEV)),
            pltpu.SemaphoreType.DMA((N_LAYERS, N_DEV)),
        ],
    )(x, Win0, Wout0, Win1, Wout1, Win2, Wout2)


# device time: 41417 ns/iter; 1.2707x vs baseline; 1.2707x over previous
import jax
import jax.numpy as jnp
from jax import lax
from jax.experimental import pallas as pl
from jax.experimental.pallas import tpu as pltpu

N_DEV = 16
N_LAYERS = 3


def kernel(x, Win0, Wout0, Win1, Wout1, Win2, Wout2):
    b, d = x.shape
    rows = b // N_DEV

    def body(
        x_ref,
        win0_ref,
        wout0_ref,
        win1_ref,
        wout1_ref,
        win2_ref,
        wout2_ref,
        out_ref,
        mine_ref,
        red_ref,
        rs_ref,
        ag_ref,
        rs_send_sems,
        rs_recv_sems,
        ag_send_sems,
        ag_recv_sems,
    ):
        my = lax.axis_index("i")
        wins = [win0_ref, win1_ref, win2_ref]
        wouts = [wout0_ref, wout1_ref, wout2_ref]

        acc = x_ref[...]
        for layer in range(N_LAYERS):
            h = jnp.maximum(
                jnp.dot(acc, wins[layer][...], preferred_element_type=jnp.float32),
                0.0,
            )
            partial = jnp.dot(
                h, wouts[layer][...], preferred_element_type=jnp.float32
            )
            mine_ref[...] = partial

            rs_sends = []
            for o in range(1, N_DEV):
                tgt = lax.rem(my + o, N_DEV)
                rdma = pltpu.make_async_remote_copy(
                    src_ref=mine_ref.at[pl.ds(tgt * rows, rows), :],
                    dst_ref=rs_ref.at[layer, o],
                    send_sem=rs_send_sems.at[layer, o],
                    recv_sem=rs_recv_sems.at[layer, o],
                    device_id=(tgt,),
                    device_id_type=pl.DeviceIdType.MESH,
                )
                rdma.start()
                rs_sends.append(rdma)

            red = mine_ref[pl.ds(my * rows, rows), :]
            for o in range(1, N_DEV):
                recv = pltpu.make_async_remote_copy(
                    src_ref=rs_ref.at[layer, o],
                    dst_ref=rs_ref.at[layer, o],
                    send_sem=rs_send_sems.at[layer, o],
                    recv_sem=rs_recv_sems.at[layer, o],
                    device_id=(my,),
                    device_id_type=pl.DeviceIdType.MESH,
                )
                recv.wait_recv()
                red = red + rs_ref[layer, o]

            red_ref[...] = red
            ag_ref[layer, pl.ds(my * rows, rows), :] = red
            ag_sends = []
            for o in range(1, N_DEV):
                tgt = lax.rem(my + o, N_DEV)
                rdma = pltpu.make_async_remote_copy(
                    src_ref=red_ref,
                    dst_ref=ag_ref.at[layer, pl.ds(my * rows, rows), :],
                    send_sem=ag_send_sems.at[layer, o],
                    recv_sem=ag_recv_sems.at[layer, o],
                    device_id=(tgt,),
                    device_id_type=pl.DeviceIdType.MESH,
                )
                rdma.start()
                ag_sends.append(rdma)

            for o in range(1, N_DEV):
                recv = pltpu.make_async_remote_copy(
                    src_ref=red_ref,
                    dst_ref=ag_ref.at[layer, pl.ds(my * rows, rows), :],
                    send_sem=ag_send_sems.at[layer, o],
                    recv_sem=ag_recv_sems.at[layer, o],
                    device_id=(my,),
                    device_id_type=pl.DeviceIdType.MESH,
                )
                recv.wait_recv()

            acc = ag_ref[layer]

            for rdma in rs_sends:
                rdma.wait_send()
            for rdma in ag_sends:
                rdma.wait_send()

        out_ref[...] = acc

    return pl.pallas_call(
        body,
        out_shape=jax.ShapeDtypeStruct((b, d), jnp.float32),
        in_specs=[pl.BlockSpec(memory_space=pltpu.VMEM)] * 7,
        out_specs=pl.BlockSpec(memory_space=pltpu.VMEM),
        scratch_shapes=[
            pltpu.VMEM((b, d), jnp.float32),
            pltpu.VMEM((rows, d), jnp.float32),
            pltpu.VMEM((N_LAYERS, N_DEV, rows, d), jnp.float32),
            pltpu.VMEM((N_LAYERS, b, d), jnp.float32),
            pltpu.SemaphoreType.DMA((N_LAYERS, N_DEV)),
            pltpu.SemaphoreType.DMA((N_LAYERS, N_DEV)),
            pltpu.SemaphoreType.DMA((N_LAYERS, N_DEV)),
            pltpu.SemaphoreType.DMA((N_LAYERS, N_DEV)),
        ],
    )(x, Win0, Wout0, Win1, Wout1, Win2, Wout2)


# device time: 38941 ns/iter; 1.3515x vs baseline; 1.0636x over previous
import jax
import jax.numpy as jnp
from jax import lax
from jax.experimental import pallas as pl
from jax.experimental.pallas import tpu as pltpu

N_DEV = 16
N_LAYERS = 3


def kernel(x, Win0, Wout0, Win1, Wout1, Win2, Wout2):
    b, d = x.shape
    rows = b // N_DEV

    def body(
        x_ref,
        win0_ref,
        wout0_ref,
        win1_ref,
        wout1_ref,
        win2_ref,
        wout2_ref,
        out_ref,
        mine_ref,
        red_ref,
        rs_ref,
        ag_ref,
        rs_send_sems,
        rs_recv_sems,
        ag_send_sems,
        ag_recv_sems,
    ):
        my = lax.axis_index("i")
        wins = [win0_ref, win1_ref, win2_ref]
        wouts = [wout0_ref, wout1_ref, wout2_ref]

        acc = x_ref[...]
        for layer in range(N_LAYERS):
            h = jnp.maximum(
                jnp.dot(acc, wins[layer][...], preferred_element_type=jnp.float32),
                0.0,
            )
            partial = jnp.dot(
                h, wouts[layer][...], preferred_element_type=jnp.float32
            )
            mine_ref[...] = partial

            if layer == 0:
                barrier_sem = pltpu.get_barrier_semaphore()
                for o in range(1, N_DEV):
                    pl.semaphore_signal(
                        barrier_sem,
                        inc=1,
                        device_id=(lax.rem(my + o, N_DEV),),
                        device_id_type=pl.DeviceIdType.MESH,
                    )
                pl.semaphore_wait(barrier_sem, N_DEV - 1)

            rs_sends = []
            for o in range(1, N_DEV):
                tgt = lax.rem(my + o, N_DEV)
                rdma = pltpu.make_async_remote_copy(
                    src_ref=mine_ref.at[pl.ds(tgt * rows, rows), :],
                    dst_ref=rs_ref.at[layer, o],
                    send_sem=rs_send_sems.at[layer, o],
                    recv_sem=rs_recv_sems.at[layer, o],
                    device_id=(tgt,),
                    device_id_type=pl.DeviceIdType.MESH,
                )
                rdma.start()
                rs_sends.append(rdma)

            red = mine_ref[pl.ds(my * rows, rows), :]
            for o in range(1, N_DEV):
                recv = pltpu.make_async_remote_copy(
                    src_ref=rs_ref.at[layer, o],
                    dst_ref=rs_ref.at[layer, o],
                    send_sem=rs_send_sems.at[layer, o],
                    recv_sem=rs_recv_sems.at[layer, o],
                    device_id=(my,),
                    device_id_type=pl.DeviceIdType.MESH,
                )
                recv.wait_recv()
                red = red + rs_ref[layer, o]

            red_ref[...] = red
            ag_ref[layer, pl.ds(my * rows, rows), :] = red
            ag_sends = []
            for o in range(1, N_DEV):
                tgt = lax.rem(my + o, N_DEV)
                rdma = pltpu.make_async_remote_copy(
                    src_ref=red_ref,
                    dst_ref=ag_ref.at[layer, pl.ds(my * rows, rows), :],
                    send_sem=ag_send_sems.at[layer, o],
                    recv_sem=ag_recv_sems.at[layer, o],
                    device_id=(tgt,),
                    device_id_type=pl.DeviceIdType.MESH,
                )
                rdma.start()
                ag_sends.append(rdma)

            for o in range(1, N_DEV):
                recv = pltpu.make_async_remote_copy(
                    src_ref=red_ref,
                    dst_ref=ag_ref.at[layer, pl.ds(my * rows, rows), :],
                    send_sem=ag_send_sems.at[layer, o],
                    recv_sem=ag_recv_sems.at[layer, o],
                    device_id=(my,),
                    device_id_type=pl.DeviceIdType.MESH,
                )
                recv.wait_recv()

            acc = ag_ref[layer]

            for rdma in rs_sends:
                rdma.wait_send()
            for rdma in ag_sends:
                rdma.wait_send()

        out_ref[...] = acc

    return pl.pallas_call(
        body,
        out_shape=jax.ShapeDtypeStruct((b, d), jnp.float32),
        in_specs=[pl.BlockSpec(memory_space=pltpu.VMEM)] * 7,
        out_specs=pl.BlockSpec(memory_space=pltpu.VMEM),
        scratch_shapes=[
            pltpu.VMEM((b, d), jnp.float32),
            pltpu.VMEM((rows, d), jnp.float32),
            pltpu.VMEM((N_LAYERS, N_DEV, rows, d), jnp.float32),
            pltpu.VMEM((N_LAYERS, b, d), jnp.float32),
            pltpu.SemaphoreType.DMA((N_LAYERS, N_DEV)),
            pltpu.SemaphoreType.DMA((N_LAYERS, N_DEV)),
            pltpu.SemaphoreType.DMA((N_LAYERS, N_DEV)),
            pltpu.SemaphoreType.DMA((N_LAYERS, N_DEV)),
        ],
        compiler_params=pltpu.CompilerParams(collective_id=0),
    )(x, Win0, Wout0, Win1, Wout1, Win2, Wout2)
